# baseline (speedup 1.0000x reference)
import jax
import jax.numpy as jnp
from jax import lax
from jax.experimental import pallas as pl
from jax.experimental.pallas import tpu as pltpu
from jax.experimental.pallas import tpu_sc as plsc

_B, _N, _K = 8, 512, 2048
_NS, _L = 16, 16
_KPT = (_B * _K) // _NS
_ITERS = _KPT // _L
_SLAB = _N * 4


def _dist_loss_body(h_ind_hbm, v_ind_hbm, lp_hbm, lc_hbm, out_hbm,
                    idx_v, tabs_v, part_v, red_v, out_v, shared, sem, sem2):
    s = lax.axis_index("s")
    b = s // 2

    khi0 = (s % 2) * 8
    cps = []
    for r in range(8):
        src = pl.ds((khi0 + r) * (_B * 128) + b * 128, 128)
        cps.append(pltpu.async_copy(
            h_ind_hbm.at[src], idx_v.at[pl.ds(r * 128, 128)], sem2))
        cps.append(pltpu.async_copy(
            v_ind_hbm.at[src], idx_v.at[pl.ds(_KPT + r * 128, 128)], sem2))
    cps.append(pltpu.async_copy(lp_hbm.at[pl.ds(b * _SLAB, _SLAB)],
                                tabs_v.at[pl.ds(0, _SLAB)], sem))
    cps.append(pltpu.async_copy(lc_hbm.at[pl.ds(b * _SLAB, _SLAB)],
                                tabs_v.at[pl.ds(_SLAB, _SLAB)], sem))
    for cp in cps:
        cp.wait()

    def make_body(base, coff):
        def body(k, carry):
            acc_t, acc_m = carry
            idx = idx_v[pl.ds(base + k * _L, _L)]
            fi = (lax.shift_right_logical(idx, 16) * 512 + coff
                  + lax.bitwise_and(lax.shift_right_logical(idx, 9), 127))
            fj = (lax.bitwise_and(lax.shift_right_logical(idx, 7), 3) * 512
                  + coff + lax.bitwise_and(idx, 127))
            pi = plsc.load_gather(tabs_v, [fi])
            pj = plsc.load_gather(tabs_v, [fj])
            gi = plsc.load_gather(tabs_v, [fi + _SLAB])
            gj = plsc.load_gather(tabs_v, [fj + _SLAB])
            dist = (pj - pi) * jnp.sign(gj - gi)
            m = (idx != 0).astype(jnp.float32)
            t = jnp.maximum(0.0, (1.0 - dist) * m)
            return acc_t + t, acc_m + m
        return body

    zero = jnp.zeros((_L,), jnp.float32)
    h_t, h_m = plsc.parallel_loop(
        0, _ITERS, unroll=4, carry=(zero, zero))(make_body(0, 256))
    v_t, v_m = plsc.parallel_loop(
        0, _ITERS, unroll=4, carry=(zero, zero))(make_body(_KPT, 0))

    part_v[pl.ds(0, _L)] = h_t
    part_v[pl.ds(_L, _L)] = h_m
    part_v[pl.ds(2 * _L, _L)] = v_t
    part_v[pl.ds(3 * _L, _L)] = v_m
    pltpu.sync_copy(part_v, shared.at[pl.ds(s * 4 * _L, 4 * _L)])
    plsc.subcore_barrier()

    @pl.when(s == 0)
    def _():
        pltpu.sync_copy(shared, red_v)

        def rbody(t, carry):
            aht, ahm, avt, avm = carry
            aht = aht + red_v[pl.ds(t * 4 * _L, _L)]
            ahm = ahm + red_v[pl.ds(t * 4 * _L + _L, _L)]
            avt = avt + red_v[pl.ds(t * 4 * _L + 2 * _L, _L)]
            avm = avm + red_v[pl.ds(t * 4 * _L + 3 * _L, _L)]
            return aht, ahm, avt, avm

        aht, ahm, avt, avm = lax.fori_loop(
            0, _NS, rbody, (zero, zero, zero, zero))
        htv = jnp.broadcast_to(jnp.sum(aht), (_L,))
        hmv = jnp.broadcast_to(jnp.sum(ahm), (_L,))
        vtv = jnp.broadcast_to(jnp.sum(avt), (_L,))
        vmv = jnp.broadcast_to(jnp.sum(avm), (_L,))
        out_v[...] = htv / (hmv + 1e-5) + vtv / (vmv + 1e-5)
        pltpu.sync_copy(out_v, out_hbm)


_launch = pl.kernel(
    _dist_loss_body,
    out_type=jax.ShapeDtypeStruct((_L,), jnp.float32),
    mesh=plsc.VectorSubcoreMesh(
        core_axis_name="c", subcore_axis_name="s",
        num_cores=1, num_subcores=_NS),
    compiler_params=pltpu.CompilerParams(needs_layout_passes=False),
    scratch_types=[
        pltpu.VMEM((2 * _KPT,), jnp.int32),
        pltpu.VMEM((2 * _SLAB,), jnp.float32),
        pltpu.VMEM((4 * _L,), jnp.float32),
        pltpu.VMEM((_NS * 4 * _L,), jnp.float32),
        pltpu.VMEM((_L,), jnp.float32),
        pltpu.VMEM_SHARED((_NS * 4 * _L,), jnp.float32),
        pltpu.SemaphoreType.DMA,
        pltpu.SemaphoreType.DMA,
    ],
)


def _phys_view(x):
    return x.reshape(_B, 4, 128, 4).transpose(0, 1, 3, 2).reshape(-1)


def _phys_view_idx(x):
    return x.reshape(_B, _K // 128, 128).transpose(1, 0, 2).reshape(-1)


@jax.jit
def kernel(h_pair_ind, v_pair_ind, logic, logi):
    out = _launch(_phys_view_idx(h_pair_ind), _phys_view_idx(v_pair_ind),
                  _phys_view(logi), _phys_view(logic))
    return out[0]


# speedup vs baseline: 1.0052x; 1.0052x over previous
import jax
import jax.numpy as jnp
from jax import lax
from jax.experimental import pallas as pl
from jax.experimental.pallas import tpu as pltpu
from jax.experimental.pallas import tpu_sc as plsc

_B, _N, _K = 8, 512, 2048
_NS, _L = 16, 16
_KPT = (_B * _K) // _NS
_ITERS = _KPT // _L
_SLAB = _N * 4


def _dist_loss_body(h_ind_hbm, v_ind_hbm, lp_hbm, lc_hbm, out_hbm,
                    idx_v, tabs_v, part_v, red_v, out_v, shared, sem, sem2):
    s = lax.axis_index("s")
    b = s // 2

    khi0 = (s % 2) * 8
    cps = []
    for r in range(8):
        src = pl.ds((khi0 + r) * (_B * 128) + b * 128, 128)
        cps.append(pltpu.async_copy(
            h_ind_hbm.at[src], idx_v.at[pl.ds(r * 128, 128)], sem2))
        cps.append(pltpu.async_copy(
            v_ind_hbm.at[src], idx_v.at[pl.ds(_KPT + r * 128, 128)], sem2))
    cps.append(pltpu.async_copy(lp_hbm.at[pl.ds(b * _SLAB, _SLAB)],
                                tabs_v.at[pl.ds(0, _SLAB)], sem))
    cps.append(pltpu.async_copy(lc_hbm.at[pl.ds(b * _SLAB, _SLAB)],
                                tabs_v.at[pl.ds(_SLAB, _SLAB)], sem))
    for cp in cps:
        cp.wait()

    def chunk(off, coff):
        idx = idx_v[pl.ds(off, _L)]
        fi = (lax.shift_right_logical(idx, 16) * 512 + coff
              + lax.bitwise_and(lax.shift_right_logical(idx, 9), 127))
        fj = (lax.bitwise_and(lax.shift_right_logical(idx, 7), 3) * 512
              + coff + lax.bitwise_and(idx, 127))
        pi = plsc.load_gather(tabs_v, [fi])
        pj = plsc.load_gather(tabs_v, [fj])
        gi = plsc.load_gather(tabs_v, [fi + _SLAB])
        gj = plsc.load_gather(tabs_v, [fj + _SLAB])
        dist = (pj - pi) * jnp.sign(gj - gi)
        m = (idx != 0).astype(jnp.float32)
        t = jnp.maximum(0.0, (1.0 - dist) * m)
        return t, m

    def make_body(base, coff):
        def body(k, carry):
            a1t, a1m, a2t, a2m = carry
            t1, m1 = chunk(base + k * _L, coff)
            t2, m2 = chunk(base + (_ITERS // 2 + k) * _L, coff)
            return a1t + t1, a1m + m1, a2t + t2, a2m + m2
        return body

    zero = jnp.zeros((_L,), jnp.float32)
    z4 = (zero, zero, zero, zero)
    h1t, h1m, h2t, h2m = lax.fori_loop(
        0, _ITERS // 2, make_body(0, 256), z4)
    v1t, v1m, v2t, v2m = lax.fori_loop(
        0, _ITERS // 2, make_body(_KPT, 0), z4)
    h_t, h_m = h1t + h2t, h1m + h2m
    v_t, v_m = v1t + v2t, v1m + v2m

    part_v[pl.ds(0, _L)] = h_t
    part_v[pl.ds(_L, _L)] = h_m
    part_v[pl.ds(2 * _L, _L)] = v_t
    part_v[pl.ds(3 * _L, _L)] = v_m
    pltpu.sync_copy(part_v, shared.at[pl.ds(s * 4 * _L, 4 * _L)])
    plsc.subcore_barrier()

    @pl.when(s == 0)
    def _():
        pltpu.sync_copy(shared, red_v)

        def rbody(t, carry):
            aht, ahm, avt, avm = carry
            aht = aht + red_v[pl.ds(t * 4 * _L, _L)]
            ahm = ahm + red_v[pl.ds(t * 4 * _L + _L, _L)]
            avt = avt + red_v[pl.ds(t * 4 * _L + 2 * _L, _L)]
            avm = avm + red_v[pl.ds(t * 4 * _L + 3 * _L, _L)]
            return aht, ahm, avt, avm

        aht, ahm, avt, avm = lax.fori_loop(
            0, _NS, rbody, (zero, zero, zero, zero))
        htv = jnp.broadcast_to(jnp.sum(aht), (_L,))
        hmv = jnp.broadcast_to(jnp.sum(ahm), (_L,))
        vtv = jnp.broadcast_to(jnp.sum(avt), (_L,))
        vmv = jnp.broadcast_to(jnp.sum(avm), (_L,))
        out_v[...] = htv / (hmv + 1e-5) + vtv / (vmv + 1e-5)
        pltpu.sync_copy(out_v, out_hbm)


_launch = pl.kernel(
    _dist_loss_body,
    out_type=jax.ShapeDtypeStruct((_L,), jnp.float32),
    mesh=plsc.VectorSubcoreMesh(
        core_axis_name="c", subcore_axis_name="s",
        num_cores=1, num_subcores=_NS),
    compiler_params=pltpu.CompilerParams(needs_layout_passes=False),
    scratch_types=[
        pltpu.VMEM((2 * _KPT,), jnp.int32),
        pltpu.VMEM((2 * _SLAB,), jnp.float32),
        pltpu.VMEM((4 * _L,), jnp.float32),
        pltpu.VMEM((_NS * 4 * _L,), jnp.float32),
        pltpu.VMEM((_L,), jnp.float32),
        pltpu.VMEM_SHARED((_NS * 4 * _L,), jnp.float32),
        pltpu.SemaphoreType.DMA,
        pltpu.SemaphoreType.DMA,
    ],
)


def _phys_view(x):
    return x.reshape(_B, 4, 128, 4).transpose(0, 1, 3, 2).reshape(-1)


def _phys_view_idx(x):
    return x.reshape(_B, _K // 128, 128).transpose(1, 0, 2).reshape(-1)


@jax.jit
def kernel(h_pair_ind, v_pair_ind, logic, logi):
    out = _launch(_phys_view_idx(h_pair_ind), _phys_view_idx(v_pair_ind),
                  _phys_view(logi), _phys_view(logic))
    return out[0]


# speedup vs baseline: 1.0192x; 1.0139x over previous
import jax
import jax.numpy as jnp
from jax import lax
from jax.experimental import pallas as pl
from jax.experimental.pallas import tpu as pltpu
from jax.experimental.pallas import tpu_sc as plsc

_B, _N, _K = 8, 512, 2048
_NS, _L = 16, 16
_KPT = (_B * _K) // _NS
_ITERS = _KPT // _L
_SLAB = _N * 4


def _dist_loss_body(h_ind_hbm, v_ind_hbm, lp_hbm, lc_hbm, out_hbm,
                    idx_v, tabs_v, part_v, red_v, out_v, shared, sem, sem2):
    s = lax.axis_index("s")
    b = s // 2

    cp0 = pltpu.async_copy(h_ind_hbm.at[pl.ds(s * _KPT, _KPT)],
                           idx_v.at[pl.ds(0, _KPT)], sem2)
    cp1 = pltpu.async_copy(v_ind_hbm.at[pl.ds(s * _KPT, _KPT)],
                           idx_v.at[pl.ds(_KPT, _KPT)], sem2)
    cp2 = pltpu.async_copy(lp_hbm.at[pl.ds(b * _SLAB, _SLAB)],
                           tabs_v.at[pl.ds(0, _SLAB)], sem)
    cp3 = pltpu.async_copy(lc_hbm.at[pl.ds(b * _SLAB, _SLAB)],
                           tabs_v.at[pl.ds(_SLAB, _SLAB)], sem)
    cp0.wait()
    cp1.wait()
    cp2.wait()
    cp3.wait()

    def make_body(base, coff):
        def body(k, carry):
            acc_t, acc_m = carry
            idx = idx_v[pl.ds(base + k * _L, _L)]
            fi = (lax.shift_right_logical(idx, 16) * 512 + coff
                  + lax.bitwise_and(lax.shift_right_logical(idx, 9), 127))
            fj = (lax.bitwise_and(lax.shift_right_logical(idx, 7), 3) * 512
                  + coff + lax.bitwise_and(idx, 127))
            pi = plsc.load_gather(tabs_v, [fi])
            pj = plsc.load_gather(tabs_v, [fj])
            gi = plsc.load_gather(tabs_v, [fi + _SLAB])
            gj = plsc.load_gather(tabs_v, [fj + _SLAB])
            dist = (pj - pi) * jnp.sign(gj - gi)
            m = (idx != 0).astype(jnp.float32)
            t = jnp.maximum(0.0, (1.0 - dist) * m)
            return acc_t + t, acc_m + m
        return body

    zero = jnp.zeros((_L,), jnp.float32)
    h_t, h_m = lax.fori_loop(0, _ITERS, make_body(0, 256), (zero, zero))
    v_t, v_m = lax.fori_loop(0, _ITERS, make_body(_KPT, 0), (zero, zero))

    part_v[pl.ds(0, _L)] = h_t
    part_v[pl.ds(_L, _L)] = h_m
    part_v[pl.ds(2 * _L, _L)] = v_t
    part_v[pl.ds(3 * _L, _L)] = v_m
    pltpu.sync_copy(part_v, shared.at[pl.ds(s * 4 * _L, 4 * _L)])
    plsc.subcore_barrier()

    @pl.when(s == 0)
    def _():
        pltpu.sync_copy(shared, red_v)

        def rbody(t, carry):
            aht, ahm, avt, avm = carry
            aht = aht + red_v[pl.ds(t * 4 * _L, _L)]
            ahm = ahm + red_v[pl.ds(t * 4 * _L + _L, _L)]
            avt = avt + red_v[pl.ds(t * 4 * _L + 2 * _L, _L)]
            avm = avm + red_v[pl.ds(t * 4 * _L + 3 * _L, _L)]
            return aht, ahm, avt, avm

        aht, ahm, avt, avm = lax.fori_loop(
            0, _NS, rbody, (zero, zero, zero, zero))
        htv = jnp.broadcast_to(jnp.sum(aht), (_L,))
        hmv = jnp.broadcast_to(jnp.sum(ahm), (_L,))
        vtv = jnp.broadcast_to(jnp.sum(avt), (_L,))
        vmv = jnp.broadcast_to(jnp.sum(avm), (_L,))
        out_v[...] = htv / (hmv + 1e-5) + vtv / (vmv + 1e-5)
        pltpu.sync_copy(out_v, out_hbm)


_launch = pl.kernel(
    _dist_loss_body,
    out_type=jax.ShapeDtypeStruct((_L,), jnp.float32),
    mesh=plsc.VectorSubcoreMesh(
        core_axis_name="c", subcore_axis_name="s",
        num_cores=1, num_subcores=_NS),
    compiler_params=pltpu.CompilerParams(needs_layout_passes=False),
    scratch_types=[
        pltpu.VMEM((2 * _KPT,), jnp.int32),
        pltpu.VMEM((2 * _SLAB,), jnp.float32),
        pltpu.VMEM((4 * _L,), jnp.float32),
        pltpu.VMEM((_NS * 4 * _L,), jnp.float32),
        pltpu.VMEM((_L,), jnp.float32),
        pltpu.VMEM_SHARED((_NS * 4 * _L,), jnp.float32),
        pltpu.SemaphoreType.DMA,
        pltpu.SemaphoreType.DMA,
    ],
)


def _phys_view(x):
    return x.reshape(_B, 4, 128, 4).transpose(0, 1, 3, 2).reshape(-1)


@jax.jit
def kernel(h_pair_ind, v_pair_ind, logic, logi):
    out = _launch(h_pair_ind.reshape(-1), v_pair_ind.reshape(-1),
                  _phys_view(logi), _phys_view(logic))
    return out[0]
